# baseline (speedup 1.0000x reference)
import functools

import jax
import jax.numpy as jnp
from jax.experimental import pallas as pl
from jax.experimental.pallas import tpu as pltpu

N_FIELDS = 60082
DIM = 64
B = 8
L = 50
BK = 1024
NBLK = (N_FIELDS + BK - 1) // BK


def _body(com_id_ref, ft_ref, fe_ref, rfe_ref, alpha_ref, nodes_ref,
          ce_ref, th_ref, ct_ref, W_proj_ref, b_proj_ref, fw_ref, fb_ref,
          cw_ref, cb_ref, w1_ref, b1_ref, w2_ref, b2_ref, out_ref):
    i = pl.program_id(0)
    w = fw_ref[0, :]
    v = jnp.sum(w[:, None] * W_proj_ref[...], axis=0)
    c0 = jnp.sum(b_proj_ref[...] * w)
    u_vec = jnp.sum(w[:, None] * w2_ref[...], axis=0)
    c2 = jnp.sum(b2_ref[...] * w)

    ft = ft_ref[...]
    s = jnp.dot(ft, v[:, None]) + c0
    g = jnp.dot(fe_ref[...], w[:, None])
    h = jax.lax.dot_general(rfe_ref[...], w1_ref[...],
                            (((1,), (1,)), ((), ()))) + b1_ref[...]
    h = jnp.where(h >= 0, h, 0.01 * h)
    m = jnp.dot(h, u_vec[:, None]) + c2
    a = alpha_ref[...]

    his = nodes_ref[0, 0, :]
    now = nodes_ref[0, 1, :]
    gidx = i * BK + jax.lax.broadcasted_iota(jnp.int32, (BK, 1), 0)
    in_now = jnp.any(gidx == now[None, :], axis=1, keepdims=True)
    in_his = jnp.any(gidx == his[None, :], axis=1, keepdims=True)
    fn = in_now.astype(jnp.float32)
    fh = in_his.astype(jnp.float32)
    mk = jnp.maximum(fn, fh)

    theta_c = th_ref[0, 0, 0]
    cstat = (1.0 - theta_c) * ce_ref[0, 0, :] + theta_c * ct_ref[0, 0, :]
    c_user = jnp.sum(cstat * cw_ref[0, :]) + cb_ref[0] + fb_ref[0]

    out = (1.0 - a * mk) * s + a * (fn * g + fh * m) + c_user
    out_ref[0, 0, :] = out[:, 0]


def kernel(company_emb, field_emb, nodes, com_id, hier_embed, raw_field_embed,
           raw_hier_embed, company_table, field_table, W_proj, b_proj, theta,
           alpha_fields, fc_field_w, fc_field_b, fc_company_w, fc_company_b,
           w1, b1, w2, b2):
    grid_spec = pltpu.PrefetchScalarGridSpec(
        num_scalar_prefetch=1,
        grid=(NBLK, B),
        in_specs=[
            pl.BlockSpec((BK, DIM), lambda i, u, ids: (i, 0)),
            pl.BlockSpec((BK, DIM), lambda i, u, ids: (i, 0)),
            pl.BlockSpec((BK, DIM), lambda i, u, ids: (i, 0)),
            pl.BlockSpec((BK, 1), lambda i, u, ids: (i, 0)),
            pl.BlockSpec((1, 2, L), lambda i, u, ids: (u, 0, 0)),
            pl.BlockSpec((1, 1, DIM), lambda i, u, ids: (u, 0, 0)),
            pl.BlockSpec((1, 1, 1), lambda i, u, ids: (ids[u], 0, 0)),
            pl.BlockSpec((1, 1, DIM), lambda i, u, ids: (ids[u], 0, 0)),
            pl.BlockSpec((DIM, DIM), lambda i, u, ids: (0, 0)),
            pl.BlockSpec((DIM,), lambda i, u, ids: (0,)),
            pl.BlockSpec((1, DIM), lambda i, u, ids: (0, 0)),
            pl.BlockSpec((1,), lambda i, u, ids: (0,)),
            pl.BlockSpec((1, DIM), lambda i, u, ids: (0, 0)),
            pl.BlockSpec((1,), lambda i, u, ids: (0,)),
            pl.BlockSpec((DIM // 2, DIM), lambda i, u, ids: (0, 0)),
            pl.BlockSpec((DIM // 2,), lambda i, u, ids: (0,)),
            pl.BlockSpec((DIM, DIM // 2), lambda i, u, ids: (0, 0)),
            pl.BlockSpec((DIM,), lambda i, u, ids: (0,)),
        ],
        out_specs=pl.BlockSpec((1, 1, BK), lambda i, u, ids: (u, 0, i)),
    )
    out = pl.pallas_call(
        _body,
        grid_spec=grid_spec,
        out_shape=jax.ShapeDtypeStruct((B, 1, N_FIELDS), jnp.float32),
    )(com_id, field_table, field_emb, raw_field_embed, alpha_fields, nodes,
      company_emb.reshape(B, 1, DIM), theta.reshape(-1, 1, 1),
      company_table.reshape(-1, 1, DIM), W_proj, b_proj, fc_field_w,
      fc_field_b, fc_company_w, fc_company_b, w1, b1, w2, b2)
    return out.reshape(B, N_FIELDS)


# speedup vs baseline: 1.0805x; 1.0805x over previous
import functools

import jax
import jax.numpy as jnp
from jax import lax
from jax.experimental import pallas as pl
from jax.experimental.pallas import tpu as pltpu
from jax.experimental.pallas import tpu_sc as plsc

N_FIELDS = 60082
DIM = 64
HID = 32
B = 8
L = 50
LP = 64
BK = 4096
NBLK = (N_FIELDS + BK - 1) // BK
NP = NBLK * BK
CHUNK = B * NP // 32


def _base_body(com_id_ref, ft_ref, ce_ref, th_ref, ct_ref, W_proj_ref,
               b_proj_ref, fw_ref, fb_ref, cw_ref, cb_ref, w2_ref, b2_ref,
               out_ref, s_ref, aux_ref):
    w = fw_ref[0, :]
    v = jnp.sum(w[:, None] * W_proj_ref[...], axis=0)
    c0 = jnp.sum(b_proj_ref[...] * w)
    s = jnp.dot(ft_ref[...], v[:, None]) + c0

    theta_c = th_ref[0, 0, 0]
    cstat = (1.0 - theta_c) * ce_ref[0, 0, :] + theta_c * ct_ref[0, 0, :]
    c_user = jnp.sum(cstat * cw_ref[0, :]) + cb_ref[0] + fb_ref[0]

    out_ref[0, 0, :] = s[:, 0] + c_user
    s_ref[0, 0, :] = s[:, 0]

    u_vec = jnp.sum(w[:, None] * w2_ref[...], axis=0)
    c2 = jnp.sum(b2_ref[...] * w)
    aux_ref[0, 0, :] = jnp.concatenate(
        [w, u_vec, jnp.full((32,), c2, jnp.float32)])


def _base_call(com_id, field_table, company_emb, theta, company_table,
               W_proj, b_proj, fc_field_w, fc_field_b, fc_company_w,
               fc_company_b, w2, b2):
    grid_spec = pltpu.PrefetchScalarGridSpec(
        num_scalar_prefetch=1,
        grid=(NBLK, B),
        in_specs=[
            pl.BlockSpec((BK, DIM), lambda i, u, ids: (i, 0)),
            pl.BlockSpec((1, 1, DIM), lambda i, u, ids: (u, 0, 0)),
            pl.BlockSpec((1, 1, 1), lambda i, u, ids: (ids[u], 0, 0)),
            pl.BlockSpec((1, 1, DIM), lambda i, u, ids: (ids[u], 0, 0)),
            pl.BlockSpec((DIM, DIM), lambda i, u, ids: (0, 0)),
            pl.BlockSpec((DIM,), lambda i, u, ids: (0,)),
            pl.BlockSpec((1, DIM), lambda i, u, ids: (0, 0)),
            pl.BlockSpec((1,), lambda i, u, ids: (0,)),
            pl.BlockSpec((1, DIM), lambda i, u, ids: (0, 0)),
            pl.BlockSpec((1,), lambda i, u, ids: (0,)),
            pl.BlockSpec((DIM, HID), lambda i, u, ids: (0, 0)),
            pl.BlockSpec((DIM,), lambda i, u, ids: (0,)),
        ],
        out_specs=[
            pl.BlockSpec((1, 1, BK), lambda i, u, ids: (u, 0, i)),
            pl.BlockSpec((1, 1, BK), lambda i, u, ids: (0, 0, i)),
            pl.BlockSpec((1, 1, 128), lambda i, u, ids: (0, 0, 0)),
        ],
    )
    return pl.pallas_call(
        _base_body,
        grid_spec=grid_spec,
        out_shape=[
            jax.ShapeDtypeStruct((B, 1, NP), jnp.float32),
            jax.ShapeDtypeStruct((1, 1, NP), jnp.float32),
            jax.ShapeDtypeStruct((1, 1, 128), jnp.float32),
        ],
    )(com_id, field_table, company_emb.reshape(B, 1, DIM),
      theta.reshape(-1, 1, 1), company_table.reshape(-1, 1, DIM),
      W_proj, b_proj, fc_field_w, fc_field_b, fc_company_w, fc_company_b,
      w2, b2)


def _full16(val):
    return jnp.full((16,), val, jnp.int32)


def _transpose_rows(rows2d, rowsT, lanes):
    for e in range(16):
        for cpt in range(4):
            chunk = rows2d[e, pl.ds(cpt * 16, 16)]
            plsc.store_scatter(rowsT, [(lanes + cpt * 16) * 16 + e], chunk)


def _corr_body(base_hbm, s_hbm, aux_hbm, fe_hbm, rfe_hbm, alpha_hbm,
               nodes_hbm, w1_hbm, b1_hbm, out_hbm, idx_v, cur_v, s_v, a_v,
               rows_v, rowsT_v, val_v, aux_v, w1_v, b1_v, sem):
    c = lax.axis_index("c")
    sid = lax.axis_index("s")
    b = c * 4 + sid // 4
    q = sid % 4
    wid = c * 16 + sid

    pltpu.sync_copy(base_hbm.at[pl.ds(wid * CHUNK, CHUNK)],
                    out_hbm.at[pl.ds(wid * CHUNK, CHUNK)])
    pltpu.sync_copy(aux_hbm, aux_v)
    pltpu.sync_copy(w1_hbm, w1_v)
    pltpu.sync_copy(b1_hbm, b1_v)
    lanes = lax.iota(jnp.int32, 16)
    plsc.subcore_barrier()

    pltpu.sync_copy(nodes_hbm.at[b, 1, pl.ds(q * 16, 16)], idx_v)
    idx = idx_v[...]
    fidx = idx + b * NP
    c1 = pltpu.async_copy(out_hbm.at[fidx], cur_v, sem)
    c2 = pltpu.async_copy(s_hbm.at[idx], s_v, sem)
    c3 = pltpu.async_copy(alpha_hbm.at[idx], a_v, sem)
    c4 = pltpu.async_copy(fe_hbm.at[idx], rows_v, sem)
    c1.wait(); c2.wait(); c3.wait(); c4.wait()
    plsc.subcore_barrier()
    _transpose_rows(rows_v, rowsT_v, lanes)

    def gbody(j, acc):
        col = rowsT_v[pl.ds(j * 16, 16)]
        wj = plsc.load_gather(aux_v, [_full16(j)])
        return acc + col * wj
    g = lax.fori_loop(0, DIM, gbody, jnp.zeros((16,), jnp.float32))
    val_v[...] = cur_v[...] + a_v[...] * (g - s_v[...])
    pltpu.async_copy(val_v, out_hbm.at[fidx], sem).wait()
    plsc.subcore_barrier()

    pltpu.sync_copy(nodes_hbm.at[b, 0, pl.ds(q * 16, 16)], idx_v)
    idx2 = idx_v[...]
    fidx2 = idx2 + b * NP
    d1 = pltpu.async_copy(out_hbm.at[fidx2], cur_v, sem)
    d2 = pltpu.async_copy(alpha_hbm.at[idx2], a_v, sem)
    d3 = pltpu.async_copy(rfe_hbm.at[idx2], rows_v, sem)
    d1.wait(); d2.wait(); d3.wait()
    plsc.subcore_barrier()
    _transpose_rows(rows_v, rowsT_v, lanes)

    def mbody(j, accs):
        col = rowsT_v[pl.ds(j * 16, 16)]
        return tuple(
            accs[k] + plsc.load_gather(w1_v, [_full16(k * DIM) + j]) * col
            for k in range(HID))
    accs = lax.fori_loop(0, DIM, mbody,
                         tuple(jnp.zeros((16,), jnp.float32)
                               for _ in range(HID)))
    m = jnp.zeros((16,), jnp.float32)
    for k in range(HID):
        hk = accs[k] + plsc.load_gather(b1_v, [_full16(k)])
        hk = jnp.where(hk >= 0, hk, 0.01 * hk)
        m = m + hk * plsc.load_gather(aux_v, [_full16(DIM + k)])
    m = m + plsc.load_gather(aux_v, [_full16(DIM + HID)])
    val_v[...] = cur_v[...] + a_v[...] * m
    pltpu.async_copy(val_v, out_hbm.at[fidx2], sem).wait()


def _corr_call(base_flat, s_flat, aux_flat, field_emb, raw_field_embed,
               alpha_flat, nodes_pad, w1, b1):
    mesh = plsc.VectorSubcoreMesh(core_axis_name="c", subcore_axis_name="s")
    f = functools.partial(
        pl.kernel,
        mesh=mesh,
        out_type=jax.ShapeDtypeStruct((B * NP,), jnp.float32),
        scratch_types=[
            pltpu.VMEM((16,), jnp.int32),
            pltpu.VMEM((16,), jnp.float32),
            pltpu.VMEM((16,), jnp.float32),
            pltpu.VMEM((16,), jnp.float32),
            pltpu.VMEM((16, DIM), jnp.float32),
            pltpu.VMEM((16 * DIM,), jnp.float32),
            pltpu.VMEM((16,), jnp.float32),
            pltpu.VMEM((128,), jnp.float32),
            pltpu.VMEM((HID * DIM,), jnp.float32),
            pltpu.VMEM((HID,), jnp.float32),
            pltpu.SemaphoreType.DMA,
        ],
        compiler_params=pltpu.CompilerParams(needs_layout_passes=False,
                                             use_tc_tiling_on_sc=False),
    )(_corr_body)
    return f(base_flat, s_flat, aux_flat, field_emb, raw_field_embed,
             alpha_flat, nodes_pad, w1, b1)


def kernel(company_emb, field_emb, nodes, com_id, hier_embed, raw_field_embed,
           raw_hier_embed, company_table, field_table, W_proj, b_proj, theta,
           alpha_fields, fc_field_w, fc_field_b, fc_company_w, fc_company_b,
           w1, b1, w2, b2):
    base, s, aux = _base_call(com_id, field_table, company_emb, theta,
                              company_table, W_proj, b_proj, fc_field_w,
                              fc_field_b, fc_company_w, fc_company_b, w2, b2)
    nodes_pad = jnp.pad(nodes.astype(jnp.int32), ((0, 0), (0, 0), (0, LP - L)),
                        mode="edge")
    out = _corr_call(base.reshape(-1), s.reshape(-1), aux.reshape(-1),
                     field_emb, raw_field_embed, alpha_fields.reshape(-1),
                     nodes_pad, w1.reshape(-1), b1)
    return out.reshape(B, NP)[:, :N_FIELDS]


# speedup vs baseline: 3.3248x; 3.0771x over previous
import functools

import jax
import jax.numpy as jnp
from jax import lax
from jax.experimental import pallas as pl
from jax.experimental.pallas import tpu as pltpu
from jax.experimental.pallas import tpu_sc as plsc

N_FIELDS = 60082
DIM = 64
HID = 32
B = 8
L = 50
LP = 64
BK = 4096
NBLK = (N_FIELDS + BK - 1) // BK
NP = NBLK * BK
CHUNK = B * NP // 32


def _small_body(com_id_ref, ce_ref, th_ref, ct_ref, Wp_ref, b_proj_ref,
                fw_ref, fwT_ref, fb_ref, cwT_ref, cb_ref, w2_ref, b2_ref,
                vrow_ref, c0_ref, c8_ref, aux_ref):
    w_row = fw_ref[...]
    w_col = fwT_ref[...]
    vrow_ref[...] = jnp.dot(w_row, Wp_ref[...])
    c0_ref[...] = jnp.dot(b_proj_ref[...], w_col)

    theta_c = th_ref[0, 0, 0]
    cstat = (1.0 - theta_c) * ce_ref[0, :, :] + theta_c * ct_ref[0, :, :]
    cu = jnp.dot(cstat, cwT_ref[...])
    c8_ref[...] = (cu + cb_ref[0] + fb_ref[0]).reshape(1, 1, 1)

    u_row = jnp.dot(w_row, w2_ref[...])
    c2 = jnp.dot(b2_ref[...], w_col)
    aux_ref[0, 0, :] = jnp.concatenate(
        [w_row[0, :], u_row[0, :], jnp.full((32,), c2[0, 0], jnp.float32)])


def _small_call(com_id, company_emb, theta, company_table, W_proj, b_proj,
                fc_field_w, fc_field_b, fc_company_w, fc_company_b, w2, b2):
    grid_spec = pltpu.PrefetchScalarGridSpec(
        num_scalar_prefetch=1,
        grid=(B,),
        in_specs=[
            pl.BlockSpec((1, 1, DIM), lambda u, ids: (u, 0, 0)),
            pl.BlockSpec((1, 1, 1), lambda u, ids: (ids[u], 0, 0)),
            pl.BlockSpec((1, 1, DIM), lambda u, ids: (ids[u], 0, 0)),
            pl.BlockSpec((DIM, DIM), lambda u, ids: (0, 0)),
            pl.BlockSpec((1, DIM), lambda u, ids: (0, 0)),
            pl.BlockSpec((1, DIM), lambda u, ids: (0, 0)),
            pl.BlockSpec((DIM, 1), lambda u, ids: (0, 0)),
            pl.BlockSpec((1,), lambda u, ids: (0,)),
            pl.BlockSpec((DIM, 1), lambda u, ids: (0, 0)),
            pl.BlockSpec((1,), lambda u, ids: (0,)),
            pl.BlockSpec((DIM, HID), lambda u, ids: (0, 0)),
            pl.BlockSpec((1, DIM), lambda u, ids: (0, 0)),
        ],
        out_specs=[
            pl.BlockSpec((1, DIM), lambda u, ids: (0, 0)),
            pl.BlockSpec((1, 1), lambda u, ids: (0, 0)),
            pl.BlockSpec((1, 1, 1), lambda u, ids: (u, 0, 0)),
            pl.BlockSpec((1, 1, 128), lambda u, ids: (0, 0, 0)),
        ],
    )
    return pl.pallas_call(
        _small_body,
        grid_spec=grid_spec,
        out_shape=[
            jax.ShapeDtypeStruct((1, DIM), jnp.float32),
            jax.ShapeDtypeStruct((1, 1), jnp.float32),
            jax.ShapeDtypeStruct((B, 1, 1), jnp.float32),
            jax.ShapeDtypeStruct((1, 1, 128), jnp.float32),
        ],
    )(com_id, company_emb.reshape(B, 1, DIM), theta.reshape(-1, 1, 1),
      company_table.reshape(-1, 1, DIM), W_proj, b_proj.reshape(1, DIM),
      fc_field_w, fc_field_w.T, fc_field_b, fc_company_w.T, fc_company_b,
      w2, b2.reshape(1, DIM))


def _base_body(ft_ref, vrow_ref, c0_ref, c8_ref, out_ref, s_ref):
    vb = jnp.broadcast_to(vrow_ref[...], (B, DIM))
    wide = jax.lax.dot_general(vb, ft_ref[...],
                               (((1,), (1,)), ((), ())))
    c0 = c0_ref[0, 0]
    ccol = c8_ref[:, 0, :]
    out_ref[:, 0, :] = wide + c0 + ccol
    s_ref[0, 0, :] = wide[0, :] + c0


def _base_call(field_table, vrow, c0, c8):
    return pl.pallas_call(
        _base_body,
        grid=(NBLK,),
        in_specs=[
            pl.BlockSpec((BK, DIM), lambda i: (i, 0)),
            pl.BlockSpec((1, DIM), lambda i: (0, 0)),
            pl.BlockSpec((1, 1), lambda i: (0, 0)),
            pl.BlockSpec((B, 1, 1), lambda i: (0, 0, 0)),
        ],
        out_specs=[
            pl.BlockSpec((B, 1, BK), lambda i: (0, 0, i)),
            pl.BlockSpec((1, 1, BK), lambda i: (0, 0, i)),
        ],
        out_shape=[
            jax.ShapeDtypeStruct((B, 1, NP), jnp.float32),
            jax.ShapeDtypeStruct((1, 1, NP), jnp.float32),
        ],
    )(field_table, vrow, c0, c8)


def _full16(val):
    return jnp.full((16,), val, jnp.int32)


def _transpose_rows(rows2d, rowsT, lanes):
    for e in range(16):
        for cpt in range(4):
            chunk = rows2d[e, pl.ds(cpt * 16, 16)]
            plsc.store_scatter(rowsT, [(lanes + cpt * 16) * 16 + e], chunk)


def _corr_body(base_hbm, s_hbm, aux_hbm, fe_hbm, rfe_hbm, alpha_hbm,
               nodes_hbm, w1_hbm, b1_hbm, out_hbm, idx_v, cur_v, s_v, a_v,
               rows_v, rowsT_v, val_v, aux_v, w1_v, b1_v, sem):
    c = lax.axis_index("c")
    sid = lax.axis_index("s")
    b = c * 4 + sid // 4
    q = sid % 4
    wid = c * 16 + sid

    pltpu.sync_copy(base_hbm.at[pl.ds(wid * CHUNK, CHUNK)],
                    out_hbm.at[pl.ds(wid * CHUNK, CHUNK)])
    pltpu.sync_copy(aux_hbm, aux_v)
    pltpu.sync_copy(w1_hbm, w1_v)
    pltpu.sync_copy(b1_hbm, b1_v)
    lanes = lax.iota(jnp.int32, 16)
    plsc.subcore_barrier()

    pltpu.sync_copy(nodes_hbm.at[b, 1, pl.ds(q * 16, 16)], idx_v)
    idx = idx_v[...]
    fidx = idx + b * NP
    c1 = pltpu.async_copy(out_hbm.at[fidx], cur_v, sem)
    c2 = pltpu.async_copy(s_hbm.at[idx], s_v, sem)
    c3 = pltpu.async_copy(alpha_hbm.at[idx], a_v, sem)
    c4 = pltpu.async_copy(fe_hbm.at[idx], rows_v, sem)
    c1.wait(); c2.wait(); c3.wait(); c4.wait()
    plsc.subcore_barrier()
    _transpose_rows(rows_v, rowsT_v, lanes)

    def gbody(j, acc):
        col = rowsT_v[pl.ds(j * 16, 16)]
        wj = plsc.load_gather(aux_v, [_full16(j)])
        return acc + col * wj
    g = lax.fori_loop(0, DIM, gbody, jnp.zeros((16,), jnp.float32))
    val_v[...] = cur_v[...] + a_v[...] * (g - s_v[...])
    pltpu.async_copy(val_v, out_hbm.at[fidx], sem).wait()
    plsc.subcore_barrier()

    pltpu.sync_copy(nodes_hbm.at[b, 0, pl.ds(q * 16, 16)], idx_v)
    idx2 = idx_v[...]
    fidx2 = idx2 + b * NP
    d1 = pltpu.async_copy(out_hbm.at[fidx2], cur_v, sem)
    d2 = pltpu.async_copy(alpha_hbm.at[idx2], a_v, sem)
    d3 = pltpu.async_copy(rfe_hbm.at[idx2], rows_v, sem)
    d1.wait(); d2.wait(); d3.wait()
    plsc.subcore_barrier()
    _transpose_rows(rows_v, rowsT_v, lanes)

    def mbody(j, accs):
        col = rowsT_v[pl.ds(j * 16, 16)]
        return tuple(
            accs[k] + plsc.load_gather(w1_v, [_full16(k * DIM) + j]) * col
            for k in range(HID))
    accs = lax.fori_loop(0, DIM, mbody,
                         tuple(jnp.zeros((16,), jnp.float32)
                               for _ in range(HID)))
    m = jnp.zeros((16,), jnp.float32)
    for k in range(HID):
        hk = accs[k] + plsc.load_gather(b1_v, [_full16(k)])
        hk = jnp.where(hk >= 0, hk, 0.01 * hk)
        m = m + hk * plsc.load_gather(aux_v, [_full16(DIM + k)])
    m = m + plsc.load_gather(aux_v, [_full16(DIM + HID)])
    val_v[...] = cur_v[...] + a_v[...] * m
    pltpu.async_copy(val_v, out_hbm.at[fidx2], sem).wait()


def _corr_call(base_flat, s_flat, aux_flat, field_emb, raw_field_embed,
               alpha_flat, nodes_pad, w1, b1):
    mesh = plsc.VectorSubcoreMesh(core_axis_name="c", subcore_axis_name="s")
    f = functools.partial(
        pl.kernel,
        mesh=mesh,
        out_type=jax.ShapeDtypeStruct((B * NP,), jnp.float32),
        scratch_types=[
            pltpu.VMEM((16,), jnp.int32),
            pltpu.VMEM((16,), jnp.float32),
            pltpu.VMEM((16,), jnp.float32),
            pltpu.VMEM((16,), jnp.float32),
            pltpu.VMEM((16, DIM), jnp.float32),
            pltpu.VMEM((16 * DIM,), jnp.float32),
            pltpu.VMEM((16,), jnp.float32),
            pltpu.VMEM((128,), jnp.float32),
            pltpu.VMEM((HID * DIM,), jnp.float32),
            pltpu.VMEM((HID,), jnp.float32),
            pltpu.SemaphoreType.DMA,
        ],
        compiler_params=pltpu.CompilerParams(needs_layout_passes=False,
                                             use_tc_tiling_on_sc=False),
    )(_corr_body)
    return f(base_flat, s_flat, aux_flat, field_emb, raw_field_embed,
             alpha_flat, nodes_pad, w1, b1)


def kernel(company_emb, field_emb, nodes, com_id, hier_embed, raw_field_embed,
           raw_hier_embed, company_table, field_table, W_proj, b_proj, theta,
           alpha_fields, fc_field_w, fc_field_b, fc_company_w, fc_company_b,
           w1, b1, w2, b2):
    vrow, c0, c8, aux = _small_call(com_id, company_emb, theta, company_table,
                                    W_proj, b_proj, fc_field_w, fc_field_b,
                                    fc_company_w, fc_company_b, w2, b2)
    base, s = _base_call(field_table, vrow, c0, c8)
    nodes_pad = jnp.pad(nodes.astype(jnp.int32), ((0, 0), (0, 0), (0, LP - L)),
                        mode="edge")
    out = _corr_call(base.reshape(-1), s.reshape(-1), aux.reshape(-1),
                     field_emb, raw_field_embed, alpha_fields.reshape(-1),
                     nodes_pad, w1.reshape(-1), b1)
    return out.reshape(B, NP)[:, :N_FIELDS]


# speedup vs baseline: 5.2639x; 1.5832x over previous
import functools

import jax
import jax.numpy as jnp
from jax import lax
from jax.experimental import pallas as pl
from jax.experimental.pallas import tpu as pltpu
from jax.experimental.pallas import tpu_sc as plsc

N_FIELDS = 60082
DIM = 64
HID = 32
B = 8
L = 50
LP = 64
BK = 4096
NBLK = (N_FIELDS + BK - 1) // BK
NP = NBLK * BK
CHUNK = B * NP // 32


def _small_body(com_id_ref, ce_ref, th_ref, ct_ref, Wp_ref, b_proj_ref,
                fw_ref, fwT_ref, fb_ref, cwT_ref, cb_ref, w2_ref, b2_ref,
                vrow_ref, c0_ref, c8_ref, urow_ref, c2_ref):
    w_row = fw_ref[...]
    w_col = fwT_ref[...]
    vrow_ref[...] = jnp.dot(w_row, Wp_ref[...])
    c0_ref[...] = jnp.dot(b_proj_ref[...], w_col)

    theta_c = th_ref[0, 0, 0]
    cstat = (1.0 - theta_c) * ce_ref[0, :, :] + theta_c * ct_ref[0, :, :]
    cu = jnp.dot(cstat, cwT_ref[...])
    c8_ref[...] = (cu + cb_ref[0] + fb_ref[0]).reshape(1, 1, 1)

    urow_ref[...] = jnp.dot(w_row, w2_ref[...])
    c2_ref[...] = jnp.dot(b2_ref[...], w_col)


def _small_call(com_id, company_emb, theta, company_table, W_proj, b_proj,
                fc_field_w, fc_field_b, fc_company_w, fc_company_b, w2, b2):
    grid_spec = pltpu.PrefetchScalarGridSpec(
        num_scalar_prefetch=1,
        grid=(B,),
        in_specs=[
            pl.BlockSpec((1, 1, DIM), lambda u, ids: (u, 0, 0)),
            pl.BlockSpec((1, 1, 1), lambda u, ids: (ids[u], 0, 0)),
            pl.BlockSpec((1, 1, DIM), lambda u, ids: (ids[u], 0, 0)),
            pl.BlockSpec((DIM, DIM), lambda u, ids: (0, 0)),
            pl.BlockSpec((1, DIM), lambda u, ids: (0, 0)),
            pl.BlockSpec((1, DIM), lambda u, ids: (0, 0)),
            pl.BlockSpec((DIM, 1), lambda u, ids: (0, 0)),
            pl.BlockSpec((1,), lambda u, ids: (0,)),
            pl.BlockSpec((DIM, 1), lambda u, ids: (0, 0)),
            pl.BlockSpec((1,), lambda u, ids: (0,)),
            pl.BlockSpec((DIM, HID), lambda u, ids: (0, 0)),
            pl.BlockSpec((1, DIM), lambda u, ids: (0, 0)),
        ],
        out_specs=[
            pl.BlockSpec((1, DIM), lambda u, ids: (0, 0)),
            pl.BlockSpec((1, 1), lambda u, ids: (0, 0)),
            pl.BlockSpec((1, 1, 1), lambda u, ids: (u, 0, 0)),
            pl.BlockSpec((1, HID), lambda u, ids: (0, 0)),
            pl.BlockSpec((1, 1), lambda u, ids: (0, 0)),
        ],
    )
    return pl.pallas_call(
        _small_body,
        grid_spec=grid_spec,
        out_shape=[
            jax.ShapeDtypeStruct((1, DIM), jnp.float32),
            jax.ShapeDtypeStruct((1, 1), jnp.float32),
            jax.ShapeDtypeStruct((B, 1, 1), jnp.float32),
            jax.ShapeDtypeStruct((1, HID), jnp.float32),
            jax.ShapeDtypeStruct((1, 1), jnp.float32),
        ],
    )(com_id, company_emb.reshape(B, 1, DIM), theta.reshape(-1, 1, 1),
      company_table.reshape(-1, 1, DIM), W_proj, b_proj.reshape(1, DIM),
      fc_field_w, fc_field_w.T, fc_field_b, fc_company_w.T, fc_company_b,
      w2, b2.reshape(1, DIM))


def _stream_body(ftT_ref, feT_ref, rfeT_ref, aT_ref, vrow_ref, c0_ref,
                 c8_ref, fw_ref, w1_ref, b1_ref, urow_ref, c2_ref,
                 out_ref, gp_ref, mp_ref):
    s_row = jnp.dot(vrow_ref[...], ftT_ref[...]) + c0_ref[0, 0]
    g_row = jnp.dot(fw_ref[...], feT_ref[...])
    h = jnp.dot(w1_ref[...], rfeT_ref[...]) + b1_ref[...]
    h = jnp.where(h >= 0, h, 0.01 * h)
    m_row = jnp.dot(urow_ref[...], h) + c2_ref[0, 0]
    a_row = aT_ref[...]
    gp_ref[0, 0, :] = (a_row * (g_row - s_row))[0, :]
    mp_ref[0, 0, :] = (a_row * m_row)[0, :]
    out_ref[:, 0, :] = s_row + c8_ref[:, 0, :]


def _stream_call(ftT, feT, rfeT, alphaT, vrow, c0, c8, fc_field_w, w1, b1col,
                 urow, c2):
    return pl.pallas_call(
        _stream_body,
        grid=(NBLK,),
        in_specs=[
            pl.BlockSpec((DIM, BK), lambda i: (0, i)),
            pl.BlockSpec((DIM, BK), lambda i: (0, i)),
            pl.BlockSpec((DIM, BK), lambda i: (0, i)),
            pl.BlockSpec((1, BK), lambda i: (0, i)),
            pl.BlockSpec((1, DIM), lambda i: (0, 0)),
            pl.BlockSpec((1, 1), lambda i: (0, 0)),
            pl.BlockSpec((B, 1, 1), lambda i: (0, 0, 0)),
            pl.BlockSpec((1, DIM), lambda i: (0, 0)),
            pl.BlockSpec((HID, DIM), lambda i: (0, 0)),
            pl.BlockSpec((HID, 1), lambda i: (0, 0)),
            pl.BlockSpec((1, HID), lambda i: (0, 0)),
            pl.BlockSpec((1, 1), lambda i: (0, 0)),
        ],
        out_specs=[
            pl.BlockSpec((B, 1, BK), lambda i: (0, 0, i)),
            pl.BlockSpec((1, 1, BK), lambda i: (0, 0, i)),
            pl.BlockSpec((1, 1, BK), lambda i: (0, 0, i)),
        ],
        out_shape=[
            jax.ShapeDtypeStruct((B, 1, NP), jnp.float32),
            jax.ShapeDtypeStruct((1, 1, NP), jnp.float32),
            jax.ShapeDtypeStruct((1, 1, NP), jnp.float32),
        ],
    )(ftT, feT, rfeT, alphaT, vrow, c0, c8, fc_field_w, w1, b1col, urow, c2)


def _corr_body(base_hbm, gp_hbm, mp_hbm, nodes_hbm, out_hbm,
               idx_v, cur_v, gm_v, val_v, sem):
    c = lax.axis_index("c")
    sid = lax.axis_index("s")
    b = c * 4 + sid // 4
    q = sid % 4
    wid = c * 16 + sid

    pltpu.sync_copy(base_hbm.at[pl.ds(wid * CHUNK, CHUNK)],
                    out_hbm.at[pl.ds(wid * CHUNK, CHUNK)])
    plsc.subcore_barrier()

    pltpu.sync_copy(nodes_hbm.at[b, 1, pl.ds(q * 16, 16)], idx_v)
    idx = idx_v[...]
    fidx = idx + b * NP
    c1 = pltpu.async_copy(out_hbm.at[fidx], cur_v, sem)
    c2 = pltpu.async_copy(gp_hbm.at[idx], gm_v, sem)
    c1.wait(); c2.wait()
    plsc.subcore_barrier()
    val_v[...] = cur_v[...] + gm_v[...]
    pltpu.async_copy(val_v, out_hbm.at[fidx], sem).wait()
    plsc.subcore_barrier()

    pltpu.sync_copy(nodes_hbm.at[b, 0, pl.ds(q * 16, 16)], idx_v)
    idx2 = idx_v[...]
    fidx2 = idx2 + b * NP
    d1 = pltpu.async_copy(out_hbm.at[fidx2], cur_v, sem)
    d2 = pltpu.async_copy(mp_hbm.at[idx2], gm_v, sem)
    d1.wait(); d2.wait()
    plsc.subcore_barrier()
    val_v[...] = cur_v[...] + gm_v[...]
    pltpu.async_copy(val_v, out_hbm.at[fidx2], sem).wait()


def _corr_call(base_flat, gp_flat, mp_flat, nodes_pad):
    mesh = plsc.VectorSubcoreMesh(core_axis_name="c", subcore_axis_name="s")
    f = functools.partial(
        pl.kernel,
        mesh=mesh,
        out_type=jax.ShapeDtypeStruct((B * NP,), jnp.float32),
        scratch_types=[
            pltpu.VMEM((16,), jnp.int32),
            pltpu.VMEM((16,), jnp.float32),
            pltpu.VMEM((16,), jnp.float32),
            pltpu.VMEM((16,), jnp.float32),
            pltpu.SemaphoreType.DMA,
        ],
        compiler_params=pltpu.CompilerParams(needs_layout_passes=False,
                                             use_tc_tiling_on_sc=False),
    )(_corr_body)
    return f(base_flat, gp_flat, mp_flat, nodes_pad)


def kernel(company_emb, field_emb, nodes, com_id, hier_embed, raw_field_embed,
           raw_hier_embed, company_table, field_table, W_proj, b_proj, theta,
           alpha_fields, fc_field_w, fc_field_b, fc_company_w, fc_company_b,
           w1, b1, w2, b2):
    vrow, c0, c8, urow, c2 = _small_call(
        com_id, company_emb, theta, company_table, W_proj, b_proj,
        fc_field_w, fc_field_b, fc_company_w, fc_company_b, w2, b2)
    base, gp, mp = _stream_call(
        field_table.T, field_emb.T, raw_field_embed.T, alpha_fields.T,
        vrow, c0, c8, fc_field_w, w1, b1.reshape(HID, 1), urow, c2)
    nodes_pad = jnp.pad(nodes.astype(jnp.int32), ((0, 0), (0, 0), (0, LP - L)),
                        mode="edge")
    out = _corr_call(base.reshape(-1), gp.reshape(-1), mp.reshape(-1),
                     nodes_pad)
    return out.reshape(B, NP)[:, :N_FIELDS]


# speedup vs baseline: 8.7843x; 1.6688x over previous
import functools

import jax
import jax.numpy as jnp
from jax import lax
from jax.experimental import pallas as pl
from jax.experimental.pallas import tpu as pltpu
from jax.experimental.pallas import tpu_sc as plsc

N_FIELDS = 60082
DIM = 64
HID = 32
B = 8
L = 50
LP = 64
BK = 4096
NBLK = (N_FIELDS + BK - 1) // BK
NP = NBLK * BK
CHUNK = B * NP // 32


def _small_body(com_id_ref, ce_ref, th_ref, ct_ref, Wp_ref, b_proj_ref,
                fw_ref, fwT_ref, fb_ref, cwT_ref, cb_ref, w2_ref, b2_ref,
                vrow_ref, c0_ref, c8_ref, urow_ref, c2_ref):
    w_row = fw_ref[...]
    w_col = fwT_ref[...]
    vrow_ref[...] = jnp.dot(w_row, Wp_ref[...])
    c0_ref[...] = jnp.dot(b_proj_ref[...], w_col)

    theta_c = th_ref[0, 0, 0]
    cstat = (1.0 - theta_c) * ce_ref[0, :, :] + theta_c * ct_ref[0, :, :]
    cu = jnp.dot(cstat, cwT_ref[...])
    c8_ref[...] = (cu + cb_ref[0] + fb_ref[0]).reshape(1, 1, 1)

    urow_ref[...] = jnp.dot(w_row, w2_ref[...])
    c2_ref[...] = jnp.dot(b2_ref[...], w_col)


def _small_call(com_id, company_emb, theta, company_table, W_proj, b_proj,
                fc_field_w, fc_field_b, fc_company_w, fc_company_b, w2, b2):
    grid_spec = pltpu.PrefetchScalarGridSpec(
        num_scalar_prefetch=1,
        grid=(B,),
        in_specs=[
            pl.BlockSpec((1, 1, DIM), lambda u, ids: (u, 0, 0)),
            pl.BlockSpec((1, 1, 1), lambda u, ids: (ids[u], 0, 0)),
            pl.BlockSpec((1, 1, DIM), lambda u, ids: (ids[u], 0, 0)),
            pl.BlockSpec((DIM, DIM), lambda u, ids: (0, 0)),
            pl.BlockSpec((1, DIM), lambda u, ids: (0, 0)),
            pl.BlockSpec((1, DIM), lambda u, ids: (0, 0)),
            pl.BlockSpec((DIM, 1), lambda u, ids: (0, 0)),
            pl.BlockSpec((1,), lambda u, ids: (0,)),
            pl.BlockSpec((DIM, 1), lambda u, ids: (0, 0)),
            pl.BlockSpec((1,), lambda u, ids: (0,)),
            pl.BlockSpec((DIM, HID), lambda u, ids: (0, 0)),
            pl.BlockSpec((1, DIM), lambda u, ids: (0, 0)),
        ],
        out_specs=[
            pl.BlockSpec((1, DIM), lambda u, ids: (0, 0)),
            pl.BlockSpec((1, 1), lambda u, ids: (0, 0)),
            pl.BlockSpec((1, 1, 1), lambda u, ids: (u, 0, 0)),
            pl.BlockSpec((1, HID), lambda u, ids: (0, 0)),
            pl.BlockSpec((1, 1), lambda u, ids: (0, 0)),
        ],
    )
    return pl.pallas_call(
        _small_body,
        grid_spec=grid_spec,
        out_shape=[
            jax.ShapeDtypeStruct((1, DIM), jnp.float32),
            jax.ShapeDtypeStruct((1, 1), jnp.float32),
            jax.ShapeDtypeStruct((B, 1, 1), jnp.float32),
            jax.ShapeDtypeStruct((1, HID), jnp.float32),
            jax.ShapeDtypeStruct((1, 1), jnp.float32),
        ],
    )(com_id, company_emb.reshape(B, 1, DIM), theta.reshape(-1, 1, 1),
      company_table.reshape(-1, 1, DIM), W_proj, b_proj.reshape(1, DIM),
      fc_field_w, fc_field_w.T, fc_field_b, fc_company_w.T, fc_company_b,
      w2, b2.reshape(1, DIM))


def _stream_body(ftT_ref, feT_ref, rfeT_ref, aT_ref, vrow_ref, c0_ref,
                 c8_ref, fw_ref, w1_ref, b1_ref, urow_ref, c2_ref,
                 out_ref, gp_ref, mp_ref):
    s_row = jnp.dot(vrow_ref[...], ftT_ref[...]) + c0_ref[0, 0]
    g_row = jnp.dot(fw_ref[...], feT_ref[...])
    h = jnp.dot(w1_ref[...], rfeT_ref[...]) + b1_ref[...]
    h = jnp.where(h >= 0, h, 0.01 * h)
    m_row = jnp.dot(urow_ref[...], h) + c2_ref[0, 0]
    a_row = aT_ref[...]
    gp_ref[0, 0, :] = (a_row * (g_row - s_row))[0, :]
    mp_ref[0, 0, :] = (a_row * m_row)[0, :]
    out_ref[:, 0, :] = s_row + c8_ref[:, 0, :]


def _stream_call(ftT, feT, rfeT, alphaT, vrow, c0, c8, fc_field_w, w1, b1col,
                 urow, c2):
    return pl.pallas_call(
        _stream_body,
        grid=(NBLK,),
        in_specs=[
            pl.BlockSpec((DIM, BK), lambda i: (0, i)),
            pl.BlockSpec((DIM, BK), lambda i: (0, i)),
            pl.BlockSpec((DIM, BK), lambda i: (0, i)),
            pl.BlockSpec((1, BK), lambda i: (0, i)),
            pl.BlockSpec((1, DIM), lambda i: (0, 0)),
            pl.BlockSpec((1, 1), lambda i: (0, 0)),
            pl.BlockSpec((B, 1, 1), lambda i: (0, 0, 0)),
            pl.BlockSpec((1, DIM), lambda i: (0, 0)),
            pl.BlockSpec((HID, DIM), lambda i: (0, 0)),
            pl.BlockSpec((HID, 1), lambda i: (0, 0)),
            pl.BlockSpec((1, HID), lambda i: (0, 0)),
            pl.BlockSpec((1, 1), lambda i: (0, 0)),
        ],
        out_specs=[
            pl.BlockSpec((B, 1, BK), lambda i: (0, 0, i)),
            pl.BlockSpec((1, 1, BK), lambda i: (0, 0, i)),
            pl.BlockSpec((1, 1, BK), lambda i: (0, 0, i)),
        ],
        out_shape=[
            jax.ShapeDtypeStruct((B, 1, NP), jnp.float32),
            jax.ShapeDtypeStruct((1, 1, NP), jnp.float32),
            jax.ShapeDtypeStruct((1, 1, NP), jnp.float32),
        ],
    )(ftT, feT, rfeT, alphaT, vrow, c0, c8, fc_field_w, w1, b1col, urow, c2)


def _corr_body(base_hbm, gp_hbm, mp_hbm, nodes_hbm, out_hbm,
               idx_v, cur_v, gm_v, val_v, buf_v, sem):
    c = lax.axis_index("c")
    sid = lax.axis_index("s")
    b = c * 4 + sid // 4
    q = sid % 4
    wid = c * 16 + sid

    pltpu.sync_copy(base_hbm.at[pl.ds(wid * CHUNK, CHUNK)], buf_v)
    pltpu.sync_copy(buf_v, out_hbm.at[pl.ds(wid * CHUNK, CHUNK)])
    plsc.subcore_barrier()

    pltpu.sync_copy(nodes_hbm.at[b, 1, pl.ds(q * 16, 16)], idx_v)
    idx = idx_v[...]
    fidx = idx + b * NP
    c1 = pltpu.async_copy(out_hbm.at[fidx], cur_v, sem)
    c2 = pltpu.async_copy(gp_hbm.at[idx], gm_v, sem)
    c1.wait(); c2.wait()
    plsc.subcore_barrier()
    val_v[...] = cur_v[...] + gm_v[...]
    pltpu.async_copy(val_v, out_hbm.at[fidx], sem).wait()
    plsc.subcore_barrier()

    pltpu.sync_copy(nodes_hbm.at[b, 0, pl.ds(q * 16, 16)], idx_v)
    idx2 = idx_v[...]
    fidx2 = idx2 + b * NP
    d1 = pltpu.async_copy(out_hbm.at[fidx2], cur_v, sem)
    d2 = pltpu.async_copy(mp_hbm.at[idx2], gm_v, sem)
    d1.wait(); d2.wait()
    plsc.subcore_barrier()
    val_v[...] = cur_v[...] + gm_v[...]
    pltpu.async_copy(val_v, out_hbm.at[fidx2], sem).wait()


def _corr_call(base_flat, gp_flat, mp_flat, nodes_pad):
    mesh = plsc.VectorSubcoreMesh(core_axis_name="c", subcore_axis_name="s")
    f = functools.partial(
        pl.kernel,
        mesh=mesh,
        out_type=jax.ShapeDtypeStruct((B * NP,), jnp.float32),
        scratch_types=[
            pltpu.VMEM((16,), jnp.int32),
            pltpu.VMEM((16,), jnp.float32),
            pltpu.VMEM((16,), jnp.float32),
            pltpu.VMEM((16,), jnp.float32),
            pltpu.VMEM((CHUNK,), jnp.float32),
            pltpu.SemaphoreType.DMA,
        ],
        compiler_params=pltpu.CompilerParams(needs_layout_passes=False,
                                             use_tc_tiling_on_sc=False),
    )(_corr_body)
    return f(base_flat, gp_flat, mp_flat, nodes_pad)


def kernel(company_emb, field_emb, nodes, com_id, hier_embed, raw_field_embed,
           raw_hier_embed, company_table, field_table, W_proj, b_proj, theta,
           alpha_fields, fc_field_w, fc_field_b, fc_company_w, fc_company_b,
           w1, b1, w2, b2):
    vrow, c0, c8, urow, c2 = _small_call(
        com_id, company_emb, theta, company_table, W_proj, b_proj,
        fc_field_w, fc_field_b, fc_company_w, fc_company_b, w2, b2)
    base, gp, mp = _stream_call(
        field_table.T, field_emb.T, raw_field_embed.T, alpha_fields.T,
        vrow, c0, c8, fc_field_w, w1, b1.reshape(HID, 1), urow, c2)
    nodes_pad = jnp.pad(nodes.astype(jnp.int32), ((0, 0), (0, 0), (0, LP - L)),
                        mode="edge")
    out = _corr_call(base.reshape(-1), gp.reshape(-1), mp.reshape(-1),
                     nodes_pad)
    return out.reshape(B, NP)[:, :N_FIELDS]


# speedup vs baseline: 9.7026x; 1.1045x over previous
import functools

import jax
import jax.numpy as jnp
from jax import lax
from jax.experimental import pallas as pl
from jax.experimental.pallas import tpu as pltpu
from jax.experimental.pallas import tpu_sc as plsc

N_FIELDS = 60082
N_COMPANY = 14695
DIM = 64
HID = 32
B = 8
L = 50
LP = 64
BK = 4096
NBLK = (N_FIELDS + BK - 1) // BK
NP = NBLK * BK
CHUNK = NP // 4


def _small_body(ceT_ref, ctT_ref, Wp_ref, b_proj_ref, fw_ref, fwT_ref,
                fb_ref, cw_ref, cb_ref, w2_ref, b2_ref,
                vrow_ref, c0_ref, urow_ref, c2_ref, q_ref, pr_ref):
    w_row = fw_ref[...]
    w_col = fwT_ref[...]
    vrow_ref[...] = jnp.dot(w_row, Wp_ref[...])
    c0_ref[...] = jnp.dot(b_proj_ref[...], w_col)
    urow_ref[...] = jnp.dot(w_row, w2_ref[...])
    c2_ref[...] = jnp.dot(b2_ref[...], w_col)

    cw_row = cw_ref[...]
    q_ref[...] = jnp.dot(cw_row, ctT_ref[...])
    p_row = jnp.dot(cw_row, ceT_ref[...])
    r = cb_ref[0] + fb_ref[0]
    pr_ref[...] = jnp.concatenate(
        [p_row, jnp.full((1, 8), r, jnp.float32)], axis=1)


def _small_call(company_emb, company_table, W_proj, b_proj, fc_field_w,
                fc_field_b, fc_company_w, fc_company_b, w2, b2):
    return pl.pallas_call(
        _small_body,
        grid=(1,),
        in_specs=[
            pl.BlockSpec((DIM, B), lambda i: (0, 0)),
            pl.BlockSpec((DIM, N_COMPANY), lambda i: (0, 0)),
            pl.BlockSpec((DIM, DIM), lambda i: (0, 0)),
            pl.BlockSpec((1, DIM), lambda i: (0, 0)),
            pl.BlockSpec((1, DIM), lambda i: (0, 0)),
            pl.BlockSpec((DIM, 1), lambda i: (0, 0)),
            pl.BlockSpec((1,), lambda i: (0,)),
            pl.BlockSpec((1, DIM), lambda i: (0, 0)),
            pl.BlockSpec((1,), lambda i: (0,)),
            pl.BlockSpec((DIM, HID), lambda i: (0, 0)),
            pl.BlockSpec((1, DIM), lambda i: (0, 0)),
        ],
        out_specs=[
            pl.BlockSpec((1, DIM), lambda i: (0, 0)),
            pl.BlockSpec((1, 1), lambda i: (0, 0)),
            pl.BlockSpec((1, HID), lambda i: (0, 0)),
            pl.BlockSpec((1, 1), lambda i: (0, 0)),
            pl.BlockSpec((1, N_COMPANY), lambda i: (0, 0)),
            pl.BlockSpec((1, 16), lambda i: (0, 0)),
        ],
        out_shape=[
            jax.ShapeDtypeStruct((1, DIM), jnp.float32),
            jax.ShapeDtypeStruct((1, 1), jnp.float32),
            jax.ShapeDtypeStruct((1, HID), jnp.float32),
            jax.ShapeDtypeStruct((1, 1), jnp.float32),
            jax.ShapeDtypeStruct((1, N_COMPANY), jnp.float32),
            jax.ShapeDtypeStruct((1, 16), jnp.float32),
        ],
    )(company_emb.T, company_table.T, W_proj, b_proj.reshape(1, DIM),
      fc_field_w, fc_field_w.T, fc_field_b, fc_company_w, fc_company_b,
      w2, b2.reshape(1, DIM))


def _stream_body(ftT_ref, feT_ref, rfeT_ref, aT_ref, vrow_ref, c0_ref,
                 fw_ref, w1_ref, b1_ref, urow_ref, c2_ref,
                 s_ref, gp_ref, mp_ref):
    s_row = jnp.dot(vrow_ref[...], ftT_ref[...]) + c0_ref[0, 0]
    g_row = jnp.dot(fw_ref[...], feT_ref[...])
    h = jnp.dot(w1_ref[...], rfeT_ref[...]) + b1_ref[...]
    h = jnp.where(h >= 0, h, 0.01 * h)
    m_row = jnp.dot(urow_ref[...], h) + c2_ref[0, 0]
    a_row = aT_ref[...]
    s_ref[0, 0, :] = s_row[0, :]
    gp_ref[0, 0, :] = (a_row * (g_row - s_row))[0, :]
    mp_ref[0, 0, :] = (a_row * m_row)[0, :]


def _stream_call(ftT, feT, rfeT, alphaT, vrow, c0, fc_field_w, w1, b1col,
                 urow, c2):
    return pl.pallas_call(
        _stream_body,
        grid=(NBLK,),
        in_specs=[
            pl.BlockSpec((DIM, BK), lambda i: (0, i)),
            pl.BlockSpec((DIM, BK), lambda i: (0, i)),
            pl.BlockSpec((DIM, BK), lambda i: (0, i)),
            pl.BlockSpec((1, BK), lambda i: (0, i)),
            pl.BlockSpec((1, DIM), lambda i: (0, 0)),
            pl.BlockSpec((1, 1), lambda i: (0, 0)),
            pl.BlockSpec((1, DIM), lambda i: (0, 0)),
            pl.BlockSpec((HID, DIM), lambda i: (0, 0)),
            pl.BlockSpec((HID, 1), lambda i: (0, 0)),
            pl.BlockSpec((1, HID), lambda i: (0, 0)),
            pl.BlockSpec((1, 1), lambda i: (0, 0)),
        ],
        out_specs=[
            pl.BlockSpec((1, 1, BK), lambda i: (0, 0, i)),
            pl.BlockSpec((1, 1, BK), lambda i: (0, 0, i)),
            pl.BlockSpec((1, 1, BK), lambda i: (0, 0, i)),
        ],
        out_shape=[
            jax.ShapeDtypeStruct((1, 1, NP), jnp.float32),
            jax.ShapeDtypeStruct((1, 1, NP), jnp.float32),
            jax.ShapeDtypeStruct((1, 1, NP), jnp.float32),
        ],
    )(ftT, feT, rfeT, alphaT, vrow, c0, fc_field_w, w1, b1col, urow, c2)


def _full16(val):
    return jnp.full((16,), val, jnp.int32)


def _corr_body(s_hbm, gp_hbm, mp_hbm, nodes_hbm, q_hbm, th_hbm, pr_hbm,
               cid_hbm, out_hbm, idx_v, cur_v, gm_v, val_v, buf_v, sem):
    c = lax.axis_index("c")
    sid = lax.axis_index("s")
    b = c * 4 + sid // 4
    q = sid % 4

    e1 = pltpu.async_copy(cid_hbm.at[_full16(b)], idx_v, sem)
    e1.wait()
    com_b = idx_v[...]
    e2 = pltpu.async_copy(th_hbm.at[com_b], cur_v, sem)
    e3 = pltpu.async_copy(q_hbm.at[com_b], gm_v, sem)
    e4 = pltpu.async_copy(pr_hbm.at[_full16(b)], val_v, sem)
    e2.wait(); e3.wait(); e4.wait()
    th = cur_v[...]
    c_user = (1.0 - th) * val_v[...] + th * gm_v[...]
    e5 = pltpu.async_copy(pr_hbm.at[_full16(8)], val_v, sem)
    e5.wait()
    c_user = c_user + val_v[...]

    pltpu.sync_copy(s_hbm.at[pl.ds(q * CHUNK, CHUNK)], buf_v)

    def add_c(j, _):
        buf_v[pl.ds(j * 16, 16)] = buf_v[pl.ds(j * 16, 16)] + c_user
        return 0
    lax.fori_loop(0, CHUNK // 16, add_c, 0)
    pltpu.sync_copy(buf_v, out_hbm.at[pl.ds(b * NP + q * CHUNK, CHUNK)])
    plsc.subcore_barrier()

    pltpu.sync_copy(nodes_hbm.at[b, 1, pl.ds(q * 16, 16)], idx_v)
    idx = idx_v[...]
    fidx = idx + b * NP
    c1 = pltpu.async_copy(out_hbm.at[fidx], cur_v, sem)
    c2 = pltpu.async_copy(gp_hbm.at[idx], gm_v, sem)
    c1.wait(); c2.wait()
    plsc.subcore_barrier()
    val_v[...] = cur_v[...] + gm_v[...]
    pltpu.async_copy(val_v, out_hbm.at[fidx], sem).wait()
    plsc.subcore_barrier()

    pltpu.sync_copy(nodes_hbm.at[b, 0, pl.ds(q * 16, 16)], idx_v)
    idx2 = idx_v[...]
    fidx2 = idx2 + b * NP
    d1 = pltpu.async_copy(out_hbm.at[fidx2], cur_v, sem)
    d2 = pltpu.async_copy(mp_hbm.at[idx2], gm_v, sem)
    d1.wait(); d2.wait()
    plsc.subcore_barrier()
    val_v[...] = cur_v[...] + gm_v[...]
    pltpu.async_copy(val_v, out_hbm.at[fidx2], sem).wait()


def _corr_call(s_flat, gp_flat, mp_flat, nodes_pad, q_flat, theta_flat,
               pr_flat, cid_pad):
    mesh = plsc.VectorSubcoreMesh(core_axis_name="c", subcore_axis_name="s")
    f = functools.partial(
        pl.kernel,
        mesh=mesh,
        out_type=jax.ShapeDtypeStruct((B * NP,), jnp.float32),
        scratch_types=[
            pltpu.VMEM((16,), jnp.int32),
            pltpu.VMEM((16,), jnp.float32),
            pltpu.VMEM((16,), jnp.float32),
            pltpu.VMEM((16,), jnp.float32),
            pltpu.VMEM((CHUNK,), jnp.float32),
            pltpu.SemaphoreType.DMA,
        ],
        compiler_params=pltpu.CompilerParams(needs_layout_passes=False,
                                             use_tc_tiling_on_sc=False),
    )(_corr_body)
    return f(s_flat, gp_flat, mp_flat, nodes_pad, q_flat, theta_flat,
             pr_flat, cid_pad)


def kernel(company_emb, field_emb, nodes, com_id, hier_embed, raw_field_embed,
           raw_hier_embed, company_table, field_table, W_proj, b_proj, theta,
           alpha_fields, fc_field_w, fc_field_b, fc_company_w, fc_company_b,
           w1, b1, w2, b2):
    vrow, c0, urow, c2, qv, pr = _small_call(
        company_emb, company_table, W_proj, b_proj, fc_field_w, fc_field_b,
        fc_company_w, fc_company_b, w2, b2)
    s, gp, mp = _stream_call(
        field_table.T, field_emb.T, raw_field_embed.T, alpha_fields.T,
        vrow, c0, fc_field_w, w1, b1.reshape(HID, 1), urow, c2)
    nodes_pad = jnp.pad(nodes.astype(jnp.int32), ((0, 0), (0, 0), (0, LP - L)),
                        mode="edge")
    cid_pad = jnp.pad(com_id.astype(jnp.int32), (0, 16 - B))
    out = _corr_call(s.reshape(-1), gp.reshape(-1), mp.reshape(-1),
                     nodes_pad, qv.reshape(-1), theta.reshape(-1),
                     pr.reshape(-1), cid_pad)
    return out.reshape(B, NP)[:, :N_FIELDS]


# speedup vs baseline: 10.6409x; 1.0967x over previous
import functools

import jax
import jax.numpy as jnp
from jax import lax
from jax.experimental import pallas as pl
from jax.experimental.pallas import tpu as pltpu
from jax.experimental.pallas import tpu_sc as plsc

N_FIELDS = 60082
N_COMPANY = 14695
DIM = 64
HID = 32
B = 8
L = 50
LP = 64
BK = 6144
NBLK = (N_FIELDS + BK - 1) // BK
NP = NBLK * BK
CHUNK = NP // 4


def _small_body(ceT_ref, ctT_ref, Wp_ref, b_proj_ref, fw_ref, fwT_ref,
                fb_ref, cw_ref, cb_ref, w2_ref, b2_ref,
                vrow_ref, c0_ref, urow_ref, c2_ref, q_ref, pr_ref):
    w_row = fw_ref[...]
    w_col = fwT_ref[...]
    vrow_ref[...] = jnp.dot(w_row, Wp_ref[...])
    c0_ref[...] = jnp.dot(b_proj_ref[...], w_col)
    urow_ref[...] = jnp.dot(w_row, w2_ref[...])
    c2_ref[...] = jnp.dot(b2_ref[...], w_col)

    cw_row = cw_ref[...]
    q_ref[...] = jnp.dot(cw_row, ctT_ref[...])
    p_row = jnp.dot(cw_row, ceT_ref[...])
    r = cb_ref[0] + fb_ref[0]
    pr_ref[...] = jnp.concatenate(
        [p_row, jnp.full((1, 8), r, jnp.float32)], axis=1)


def _small_call(company_emb, company_table, W_proj, b_proj, fc_field_w,
                fc_field_b, fc_company_w, fc_company_b, w2, b2):
    return pl.pallas_call(
        _small_body,
        grid=(1,),
        in_specs=[
            pl.BlockSpec((DIM, B), lambda i: (0, 0)),
            pl.BlockSpec((DIM, N_COMPANY), lambda i: (0, 0)),
            pl.BlockSpec((DIM, DIM), lambda i: (0, 0)),
            pl.BlockSpec((1, DIM), lambda i: (0, 0)),
            pl.BlockSpec((1, DIM), lambda i: (0, 0)),
            pl.BlockSpec((DIM, 1), lambda i: (0, 0)),
            pl.BlockSpec((1,), lambda i: (0,)),
            pl.BlockSpec((1, DIM), lambda i: (0, 0)),
            pl.BlockSpec((1,), lambda i: (0,)),
            pl.BlockSpec((DIM, HID), lambda i: (0, 0)),
            pl.BlockSpec((1, DIM), lambda i: (0, 0)),
        ],
        out_specs=[
            pl.BlockSpec((1, DIM), lambda i: (0, 0)),
            pl.BlockSpec((1, 1), lambda i: (0, 0)),
            pl.BlockSpec((1, HID), lambda i: (0, 0)),
            pl.BlockSpec((1, 1), lambda i: (0, 0)),
            pl.BlockSpec((1, N_COMPANY), lambda i: (0, 0)),
            pl.BlockSpec((1, 16), lambda i: (0, 0)),
        ],
        out_shape=[
            jax.ShapeDtypeStruct((1, DIM), jnp.float32),
            jax.ShapeDtypeStruct((1, 1), jnp.float32),
            jax.ShapeDtypeStruct((1, HID), jnp.float32),
            jax.ShapeDtypeStruct((1, 1), jnp.float32),
            jax.ShapeDtypeStruct((1, N_COMPANY), jnp.float32),
            jax.ShapeDtypeStruct((1, 16), jnp.float32),
        ],
    )(company_emb.T, company_table.T, W_proj, b_proj.reshape(1, DIM),
      fc_field_w, fc_field_w.T, fc_field_b, fc_company_w, fc_company_b,
      w2, b2.reshape(1, DIM))


def _stream_body(ftT_ref, feT_ref, rfeT_ref, aT_ref, vrow_ref, c0_ref,
                 fw_ref, w1_ref, b1_ref, urow_ref, c2_ref,
                 s_ref, gp_ref, mp_ref):
    s_row = jnp.dot(vrow_ref[...], ftT_ref[...]) + c0_ref[0, 0]
    g_row = jnp.dot(fw_ref[...], feT_ref[...])
    h = jnp.dot(w1_ref[...], rfeT_ref[...]) + b1_ref[...]
    h = jnp.where(h >= 0, h, 0.01 * h)
    m_row = jnp.dot(urow_ref[...], h) + c2_ref[0, 0]
    a_row = aT_ref[...]
    s_ref[0, 0, :] = s_row[0, :]
    gp_ref[0, 0, :] = (a_row * (g_row - s_row))[0, :]
    mp_ref[0, 0, :] = (a_row * m_row)[0, :]


def _stream_call(ftT, feT, rfeT, alphaT, vrow, c0, fc_field_w, w1, b1col,
                 urow, c2):
    return pl.pallas_call(
        _stream_body,
        grid=(NBLK,),
        in_specs=[
            pl.BlockSpec((DIM, BK), lambda i: (0, i)),
            pl.BlockSpec((DIM, BK), lambda i: (0, i)),
            pl.BlockSpec((DIM, BK), lambda i: (0, i)),
            pl.BlockSpec((1, BK), lambda i: (0, i)),
            pl.BlockSpec((1, DIM), lambda i: (0, 0)),
            pl.BlockSpec((1, 1), lambda i: (0, 0)),
            pl.BlockSpec((1, DIM), lambda i: (0, 0)),
            pl.BlockSpec((HID, DIM), lambda i: (0, 0)),
            pl.BlockSpec((HID, 1), lambda i: (0, 0)),
            pl.BlockSpec((1, HID), lambda i: (0, 0)),
            pl.BlockSpec((1, 1), lambda i: (0, 0)),
        ],
        out_specs=[
            pl.BlockSpec((1, 1, BK), lambda i: (0, 0, i)),
            pl.BlockSpec((1, 1, BK), lambda i: (0, 0, i)),
            pl.BlockSpec((1, 1, BK), lambda i: (0, 0, i)),
        ],
        out_shape=[
            jax.ShapeDtypeStruct((1, 1, NP), jnp.float32),
            jax.ShapeDtypeStruct((1, 1, NP), jnp.float32),
            jax.ShapeDtypeStruct((1, 1, NP), jnp.float32),
        ],
    )(ftT, feT, rfeT, alphaT, vrow, c0, fc_field_w, w1, b1col, urow, c2)


def _full16(val):
    return jnp.full((16,), val, jnp.int32)


def _corr_body(s_hbm, gp_hbm, mp_hbm, nodes_hbm, q_hbm, th_hbm, pr_hbm,
               cid_hbm, out_hbm, idx_v, cur_v, gm_v, val_v, buf_v, sem):
    c = lax.axis_index("c")
    sid = lax.axis_index("s")
    b = c * 4 + sid // 4
    q = sid % 4

    e1 = pltpu.async_copy(cid_hbm.at[_full16(b)], idx_v, sem)
    e1.wait()
    com_b = idx_v[...]
    e2 = pltpu.async_copy(th_hbm.at[com_b], cur_v, sem)
    e3 = pltpu.async_copy(q_hbm.at[com_b], gm_v, sem)
    e4 = pltpu.async_copy(pr_hbm.at[_full16(b)], val_v, sem)
    e2.wait(); e3.wait(); e4.wait()
    th = cur_v[...]
    c_user = (1.0 - th) * val_v[...] + th * gm_v[...]
    e5 = pltpu.async_copy(pr_hbm.at[_full16(8)], val_v, sem)
    e5.wait()
    c_user = c_user + val_v[...]

    pltpu.sync_copy(s_hbm.at[pl.ds(q * CHUNK, CHUNK)], buf_v)

    def add_c(j, _):
        for u in range(8):
            off = j * 128 + u * 16
            buf_v[pl.ds(off, 16)] = buf_v[pl.ds(off, 16)] + c_user
        return 0
    lax.fori_loop(0, CHUNK // 128, add_c, 0)
    pltpu.sync_copy(buf_v, out_hbm.at[pl.ds(b * NP + q * CHUNK, CHUNK)])
    plsc.subcore_barrier()

    pltpu.sync_copy(nodes_hbm.at[b, 1, pl.ds(q * 16, 16)], idx_v)
    idx = idx_v[...]
    fidx = idx + b * NP
    c1 = pltpu.async_copy(out_hbm.at[fidx], cur_v, sem)
    c2 = pltpu.async_copy(gp_hbm.at[idx], gm_v, sem)
    c1.wait(); c2.wait()
    plsc.subcore_barrier()
    val_v[...] = cur_v[...] + gm_v[...]
    pltpu.async_copy(val_v, out_hbm.at[fidx], sem).wait()
    plsc.subcore_barrier()

    pltpu.sync_copy(nodes_hbm.at[b, 0, pl.ds(q * 16, 16)], idx_v)
    idx2 = idx_v[...]
    fidx2 = idx2 + b * NP
    d1 = pltpu.async_copy(out_hbm.at[fidx2], cur_v, sem)
    d2 = pltpu.async_copy(mp_hbm.at[idx2], gm_v, sem)
    d1.wait(); d2.wait()
    plsc.subcore_barrier()
    val_v[...] = cur_v[...] + gm_v[...]
    pltpu.async_copy(val_v, out_hbm.at[fidx2], sem).wait()


def _corr_call(s_flat, gp_flat, mp_flat, nodes_pad, q_flat, theta_flat,
               pr_flat, cid_pad):
    mesh = plsc.VectorSubcoreMesh(core_axis_name="c", subcore_axis_name="s")
    f = functools.partial(
        pl.kernel,
        mesh=mesh,
        out_type=jax.ShapeDtypeStruct((B * NP,), jnp.float32),
        scratch_types=[
            pltpu.VMEM((16,), jnp.int32),
            pltpu.VMEM((16,), jnp.float32),
            pltpu.VMEM((16,), jnp.float32),
            pltpu.VMEM((16,), jnp.float32),
            pltpu.VMEM((CHUNK,), jnp.float32),
            pltpu.SemaphoreType.DMA,
        ],
        compiler_params=pltpu.CompilerParams(needs_layout_passes=False,
                                             use_tc_tiling_on_sc=False),
    )(_corr_body)
    return f(s_flat, gp_flat, mp_flat, nodes_pad, q_flat, theta_flat,
             pr_flat, cid_pad)


def kernel(company_emb, field_emb, nodes, com_id, hier_embed, raw_field_embed,
           raw_hier_embed, company_table, field_table, W_proj, b_proj, theta,
           alpha_fields, fc_field_w, fc_field_b, fc_company_w, fc_company_b,
           w1, b1, w2, b2):
    vrow, c0, urow, c2, qv, pr = _small_call(
        company_emb, company_table, W_proj, b_proj, fc_field_w, fc_field_b,
        fc_company_w, fc_company_b, w2, b2)
    s, gp, mp = _stream_call(
        field_table.T, field_emb.T, raw_field_embed.T, alpha_fields.T,
        vrow, c0, fc_field_w, w1, b1.reshape(HID, 1), urow, c2)
    nodes_pad = jnp.pad(nodes.astype(jnp.int32), ((0, 0), (0, 0), (0, LP - L)),
                        mode="edge")
    cid_pad = jnp.pad(com_id.astype(jnp.int32), (0, 16 - B))
    out = _corr_call(s.reshape(-1), gp.reshape(-1), mp.reshape(-1),
                     nodes_pad, qv.reshape(-1), theta.reshape(-1),
                     pr.reshape(-1), cid_pad)
    return out.reshape(B, NP)[:, :N_FIELDS]
